# baseline (speedup 1.0000x reference)
import functools

import jax
import jax.numpy as jnp
from jax import lax
from jax.experimental import pallas as pl
from jax.experimental.pallas import tpu as pltpu
from jax.experimental.pallas import tpu_sc as plsc

_NC = 2
_NS = 16
_CH = 80


def _expand_kernel(x_ref, g_ref, be_ref, mu_ref, var_ref, w_ref, y_ref):
    eps = 1e-5
    x = x_ref[...]
    scale = g_ref[...] * lax.rsqrt(var_ref[...] + eps)
    h = (x - mu_ref[...]) * scale + be_ref[...]
    h = h * (1.0 / (1.0 + jnp.exp(-h)))
    h16 = h.astype(jnp.bfloat16)
    for k in range(w_ref.shape[0]):
        y_ref[k] = jnp.dot(h16, w_ref[k].astype(jnp.bfloat16),
                           preferred_element_type=jnp.float32)


def _expand_y(x, bn_gamma, bn_beta, bn_mean, bn_var, W):
    n, f_in = x.shape
    kvol, _, f_out = W.shape
    bn = 400
    grid = (n // bn,)
    return pl.pallas_call(
        _expand_kernel,
        grid=grid,
        in_specs=[
            pl.BlockSpec((bn, f_in), lambda i: (i, 0)),
            pl.BlockSpec((1, f_in), lambda i: (0, 0)),
            pl.BlockSpec((1, f_in), lambda i: (0, 0)),
            pl.BlockSpec((1, f_in), lambda i: (0, 0)),
            pl.BlockSpec((1, f_in), lambda i: (0, 0)),
            pl.BlockSpec((kvol, f_in, f_out), lambda i: (0, 0, 0)),
        ],
        out_specs=pl.BlockSpec((kvol, bn, f_out), lambda i: (0, i, 0)),
        out_shape=jax.ShapeDtypeStruct((kvol, n, f_out), jnp.float32),
    )(x, bn_gamma.reshape(1, -1), bn_beta.reshape(1, -1),
      bn_mean.reshape(1, -1), bn_var.reshape(1, -1), W)


def _sc_gather_scatter(y, src, dst, kid, n_nodes):
    e = src.shape[0]
    f = y.shape[1]
    nw = _NC * _NS
    epw = e // nw
    nch = epw // _CH
    rch = 80
    nrch = n_nodes // rch

    mesh = plsc.VectorSubcoreMesh(core_axis_name="c", subcore_axis_name="s")

    @functools.partial(
        pl.kernel, mesh=mesh,
        out_type=jax.ShapeDtypeStruct((_NC * n_nodes, f), jnp.float32),
        scratch_types=[
            pltpu.VMEM((epw,), jnp.int32),
            pltpu.VMEM((epw,), jnp.int32),
            pltpu.VMEM((_CH,), jnp.int32),
            pltpu.VMEM((_CH,), jnp.int32),
            pltpu.VMEM((_CH,), jnp.int32),
            pltpu.VMEM((_CH,), jnp.int32),
            pltpu.VMEM((_CH, f), jnp.float32),
            pltpu.VMEM((_CH, f), jnp.float32),
            pltpu.VMEM_SHARED((n_nodes, f), jnp.float32),
            pltpu.SemaphoreType.DMA,
            pltpu.SemaphoreType.DMA,
            pltpu.SemaphoreType.DMA,
            pltpu.SemaphoreType.DMA,
        ],
    )
    def body(y_hbm, src_hbm, dst_hbm, kid_hbm, zero_hbm, out_hbm,
             src_v, kid_v, gidx0_v, gidx1_v, sidx0_v, sidx1_v,
             rows0_v, rows1_v, acc_sh, sem0, sem1, semd0, semd1):
        c = lax.axis_index("c")
        s = lax.axis_index("s")
        wid = c * _NS + s
        for t in range((nrch + _NS - 1) // _NS):
            rj = s + _NS * t
            @pl.when(rj < nrch)
            def _():
                pltpu.sync_copy(zero_hbm.at[pl.ds(rj * rch, rch)],
                                acc_sh.at[pl.ds(rj * rch, rch)])
        base_e = wid * epw
        pltpu.sync_copy(src_hbm.at[pl.ds(base_e, epw)], src_v)
        pltpu.sync_copy(kid_hbm.at[pl.ds(base_e, epw)], kid_v)
        plsc.subcore_barrier()

        def compute_idx(j, gidx_v):
            base = j * _CH
            for i in range(_CH // 16):
                o = base + i * 16
                s16 = src_v[pl.ds(o, 16)]
                k16 = kid_v[pl.ds(o, 16)]
                gidx_v[pl.ds(i * 16, 16)] = k16 * n_nodes + s16

        def fire(gidx_v, rows_v, sem):
            pltpu.async_copy(y_hbm.at[gidx_v], rows_v, sem)

        def drain(gidx_v, rows_v, sem):
            pltpu.make_async_copy(y_hbm.at[gidx_v], rows_v, sem).wait()

        def fire_d(j, sidx_v, semd):
            pltpu.async_copy(dst_hbm.at[pl.ds(base_e + j * _CH, _CH)],
                             sidx_v, semd)

        def drain_d(sidx_v, semd):
            pltpu.make_async_copy(dst_hbm.at[pl.ds(base_e, _CH)],
                                  sidx_v, semd).wait()

        def scat(rows_v, sidx_v):
            pltpu.sync_copy(rows_v, acc_sh.at[sidx_v], add=True)

        compute_idx(0, gidx0_v)
        fire(gidx0_v, rows0_v, sem0)
        fire_d(0, sidx0_v, semd0)

        def pair(t, carry):
            j0 = 2 * t
            compute_idx(j0 + 1, gidx1_v)
            drain(gidx0_v, rows0_v, sem0)
            fire(gidx1_v, rows1_v, sem1)
            fire_d(j0 + 1, sidx1_v, semd1)
            drain_d(sidx0_v, semd0)
            scat(rows0_v, sidx0_v)
            compute_idx(j0 + 2, gidx0_v)
            drain(gidx1_v, rows1_v, sem1)
            fire(gidx0_v, rows0_v, sem0)
            fire_d(j0 + 2, sidx0_v, semd0)
            drain_d(sidx1_v, semd1)
            scat(rows1_v, sidx1_v)
            return carry

        lax.fori_loop(0, (nch - 1) // 2, pair, 0)
        drain(gidx0_v, rows0_v, sem0)
        drain_d(sidx0_v, semd0)
        scat(rows0_v, sidx0_v)
        plsc.subcore_barrier()
        for t in range((nrch + _NS - 1) // _NS):
            rj = s + _NS * t
            @pl.when(rj < nrch)
            def _():
                pltpu.sync_copy(acc_sh.at[pl.ds(rj * rch, rch)],
                                out_hbm.at[pl.ds(c * n_nodes + rj * rch, rch)])

    zeros = jnp.zeros((n_nodes, f), jnp.float32)
    return body(y, src, dst, kid, zeros)


def _combine_kernel(p_ref, b_ref, o_ref):
    o_ref[...] = p_ref[0] + p_ref[1] + b_ref[...]


def _combine(parts, b, n_nodes, f_out):
    bn = 1000
    grid = (n_nodes // bn,)
    return pl.pallas_call(
        _combine_kernel,
        grid=grid,
        in_specs=[
            pl.BlockSpec((2, bn, f_out), lambda i: (0, i, 0)),
            pl.BlockSpec((1, f_out), lambda i: (0, 0)),
        ],
        out_specs=pl.BlockSpec((bn, f_out), lambda i: (i, 0)),
        out_shape=jax.ShapeDtypeStruct((n_nodes, f_out), jnp.float32),
    )(parts, b.reshape(1, -1))


def kernel(x, edge_index, kernel_id, bn_gamma, bn_beta, bn_mean, bn_var, W, b):
    n, _ = x.shape
    kvol, _, f_out = W.shape
    y = _expand_y(x, bn_gamma, bn_beta, bn_mean, bn_var, W)
    y = y.reshape(kvol * n, f_out)
    src = edge_index[0]
    dst = edge_index[1]
    parts = _sc_gather_scatter(y, src, dst, kernel_id, n)
    return _combine(parts.reshape(2, n, f_out), b, n, f_out)


# speedup vs baseline: 1.1965x; 1.1965x over previous
import functools

import jax
import jax.numpy as jnp
from jax import lax
from jax.experimental import pallas as pl
from jax.experimental.pallas import tpu as pltpu
from jax.experimental.pallas import tpu_sc as plsc

_NC = 2
_NS = 16
_CH = 80


def _expand_kernel(x_ref, g_ref, be_ref, mu_ref, var_ref, w_ref, y_ref):
    eps = 1e-5
    x = x_ref[...]
    scale = g_ref[...] * lax.rsqrt(var_ref[...] + eps)
    h = (x - mu_ref[...]) * scale + be_ref[...]
    h = h * (1.0 / (1.0 + jnp.exp(-h)))
    for k in range(w_ref.shape[0]):
        y_ref[k] = jnp.dot(h, w_ref[k], preferred_element_type=jnp.float32)


def _expand_y(x, bn_gamma, bn_beta, bn_mean, bn_var, W):
    n, f_in = x.shape
    kvol, _, f_out = W.shape
    bn = 400
    grid = (n // bn,)
    return pl.pallas_call(
        _expand_kernel,
        grid=grid,
        in_specs=[
            pl.BlockSpec((bn, f_in), lambda i: (i, 0)),
            pl.BlockSpec((1, f_in), lambda i: (0, 0)),
            pl.BlockSpec((1, f_in), lambda i: (0, 0)),
            pl.BlockSpec((1, f_in), lambda i: (0, 0)),
            pl.BlockSpec((1, f_in), lambda i: (0, 0)),
            pl.BlockSpec((kvol, f_in, f_out), lambda i: (0, 0, 0)),
        ],
        out_specs=pl.BlockSpec((kvol, bn, f_out), lambda i: (0, i, 0)),
        out_shape=jax.ShapeDtypeStruct((kvol, n, f_out), jnp.float32),
    )(x, bn_gamma.reshape(1, -1), bn_beta.reshape(1, -1),
      bn_mean.reshape(1, -1), bn_var.reshape(1, -1), W)


def _sc_gather_scatter(y, src, dst, kid, n_nodes):
    e = src.shape[0]
    f = y.shape[1]
    nw = _NC * _NS
    epw = e // nw
    nch = epw // _CH
    rch = 80
    nrch = n_nodes // rch

    mesh = plsc.VectorSubcoreMesh(core_axis_name="c", subcore_axis_name="s")

    @functools.partial(
        pl.kernel, mesh=mesh,
        out_type=jax.ShapeDtypeStruct((_NC * n_nodes, f), jnp.float32),
        scratch_types=[
            pltpu.VMEM((epw,), jnp.int32),
            pltpu.VMEM((epw,), jnp.int32),
            pltpu.VMEM((_CH,), jnp.int32),
            pltpu.VMEM((_CH,), jnp.int32),
            pltpu.VMEM((_CH,), jnp.int32),
            pltpu.VMEM((_CH,), jnp.int32),
            pltpu.VMEM((_CH, f), jnp.float32),
            pltpu.VMEM((_CH, f), jnp.float32),
            pltpu.VMEM_SHARED((n_nodes, f), jnp.float32),
            pltpu.SemaphoreType.DMA,
            pltpu.SemaphoreType.DMA,
            pltpu.SemaphoreType.DMA,
            pltpu.SemaphoreType.DMA,
        ],
    )
    def body(y_hbm, src_hbm, dst_hbm, kid_hbm, zero_hbm, out_hbm,
             src_v, kid_v, gidx0_v, gidx1_v, sidx0_v, sidx1_v,
             rows0_v, rows1_v, acc_sh, sem0, sem1, semd0, semd1):
        c = lax.axis_index("c")
        s = lax.axis_index("s")
        wid = c * _NS + s
        for t in range((nrch + _NS - 1) // _NS):
            rj = s + _NS * t
            @pl.when(rj < nrch)
            def _():
                pltpu.sync_copy(zero_hbm.at[pl.ds(rj * rch, rch)],
                                acc_sh.at[pl.ds(rj * rch, rch)])
        base_e = wid * epw
        pltpu.sync_copy(src_hbm.at[pl.ds(base_e, epw)], src_v)
        pltpu.sync_copy(kid_hbm.at[pl.ds(base_e, epw)], kid_v)
        plsc.subcore_barrier()

        def compute_idx(j, gidx_v):
            base = j * _CH
            for i in range(_CH // 16):
                o = base + i * 16
                s16 = src_v[pl.ds(o, 16)]
                k16 = kid_v[pl.ds(o, 16)]
                gidx_v[pl.ds(i * 16, 16)] = k16 * n_nodes + s16

        def fire(gidx_v, rows_v, sem):
            pltpu.async_copy(y_hbm.at[gidx_v], rows_v, sem)

        def drain(gidx_v, rows_v, sem):
            pltpu.make_async_copy(y_hbm.at[gidx_v], rows_v, sem).wait()

        def fire_d(j, sidx_v, semd):
            pltpu.async_copy(dst_hbm.at[pl.ds(base_e + j * _CH, _CH)],
                             sidx_v, semd)

        def drain_d(sidx_v, semd):
            pltpu.make_async_copy(dst_hbm.at[pl.ds(base_e, _CH)],
                                  sidx_v, semd).wait()

        def scat(rows_v, sidx_v):
            pltpu.sync_copy(rows_v, acc_sh.at[sidx_v], add=True)

        compute_idx(0, gidx0_v)
        fire(gidx0_v, rows0_v, sem0)
        fire_d(0, sidx0_v, semd0)

        def pair(t, carry):
            j0 = 2 * t
            compute_idx(j0 + 1, gidx1_v)
            fire(gidx1_v, rows1_v, sem1)
            fire_d(j0 + 1, sidx1_v, semd1)
            drain(gidx0_v, rows0_v, sem0)
            drain_d(sidx0_v, semd0)
            scat(rows0_v, sidx0_v)
            compute_idx(j0 + 2, gidx0_v)
            fire(gidx0_v, rows0_v, sem0)
            fire_d(j0 + 2, sidx0_v, semd0)
            drain(gidx1_v, rows1_v, sem1)
            drain_d(sidx1_v, semd1)
            scat(rows1_v, sidx1_v)
            return carry

        lax.fori_loop(0, (nch - 1) // 2, pair, 0)
        drain(gidx0_v, rows0_v, sem0)
        drain_d(sidx0_v, semd0)
        scat(rows0_v, sidx0_v)
        plsc.subcore_barrier()
        for t in range((nrch + _NS - 1) // _NS):
            rj = s + _NS * t
            @pl.when(rj < nrch)
            def _():
                pltpu.sync_copy(acc_sh.at[pl.ds(rj * rch, rch)],
                                out_hbm.at[pl.ds(c * n_nodes + rj * rch, rch)])

    zeros = jnp.zeros((n_nodes, f), jnp.float32)
    return body(y, src, dst, kid, zeros)


def _combine_kernel(p_ref, b_ref, o_ref):
    o_ref[...] = p_ref[0] + p_ref[1] + b_ref[...]


def _combine(parts, b, n_nodes, f_out):
    bn = 1000
    grid = (n_nodes // bn,)
    return pl.pallas_call(
        _combine_kernel,
        grid=grid,
        in_specs=[
            pl.BlockSpec((2, bn, f_out), lambda i: (0, i, 0)),
            pl.BlockSpec((1, f_out), lambda i: (0, 0)),
        ],
        out_specs=pl.BlockSpec((bn, f_out), lambda i: (i, 0)),
        out_shape=jax.ShapeDtypeStruct((n_nodes, f_out), jnp.float32),
    )(parts, b.reshape(1, -1))


def kernel(x, edge_index, kernel_id, bn_gamma, bn_beta, bn_mean, bn_var, W, b):
    n, _ = x.shape
    kvol, _, f_out = W.shape
    y = _expand_y(x, bn_gamma, bn_beta, bn_mean, bn_var, W)
    y = y.reshape(kvol * n, f_out)
    src = edge_index[0]
    dst = edge_index[1]
    parts = _sc_gather_scatter(y, src, dst, kernel_id, n)
    return _combine(parts.reshape(2, n, f_out), b, n, f_out)
